# baseline (speedup 1.0000x reference)
import jax
import jax.numpy as jnp
from jax.experimental import pallas as pl
from jax.experimental.pallas import tpu as pltpu

B = 1024
D = 128
C = 2000
CP = 2048
K = 10
BB = 256
NBLK = B // BB
NCH = CP // 128

_DN_T = (((1,), (1,)), ((), ()))
_DN_ROW = (((1,), (0,)), ((), ()))
_DN_COL = (((0,), (0,)), ((), ()))
_BIG = jnp.iinfo(jnp.int32).max


def _fold_lanes(x, op):
    m = x[:, 0:128]
    for k in range(1, NCH):
        m = op(m, x[:, k * 128:(k + 1) * 128])
    return m


def _min_lanes(x):
    return jnp.min(_fold_lanes(x, jnp.minimum), axis=1, keepdims=True)


def _max_lanes(x):
    return jnp.max(_fold_lanes(x, jnp.maximum), axis=1, keepdims=True)


def _first_index_where(cond, iota):
    cand = None
    for k in range(NCH):
        sl = slice(k * 128, (k + 1) * 128)
        c = jnp.where(cond[:, sl], iota[:, sl], _BIG)
        cand = c if cand is None else jnp.minimum(cand, c)
    return jnp.min(cand, axis=1, keepdims=True)


def _body(data_ref, labels_ref, n_ref, mu_ref, s_ref, cl_ref,
          scores_ref, pred_ref, clusters_ref, nnew_ref, munew_ref, snew_ref,
          iv_ref, muiv_ref, t3_ref, assign_ref, claug_ref, count_ref,
          j_ref, muaug_ref):
    i = pl.program_id(0)

    @pl.when(i == 0)
    def _init():
        var = s_ref[:] / jnp.maximum(n_ref[:], 1.0)[:, None] + 1e-6
        iv = 1.0 / var
        iv_ref[:] = iv
        muiv_ref[:] = mu_ref[:] * iv
        t3_ref[:] = jnp.sum(mu_ref[:] * mu_ref[:] * iv, axis=1)[None, :]
        cl = cl_ref[:]
        m = jnp.max(cl, axis=1, keepdims=True)
        cidx = jax.lax.broadcasted_iota(jnp.int32, cl.shape, 1)
        assign_ref[:] = jnp.min(jnp.where(cl == m, cidx, _BIG), axis=1)[None, :]
        claug_ref[:] = cl.astype(jnp.float32)
        count_ref[:] = jnp.zeros_like(count_ref)

    @pl.when(i < NBLK)
    def _activation():
        b = i
        x = data_ref[:]
        t1 = jax.lax.dot_general(x * x, iv_ref[:], _DN_T,
                                 preferred_element_type=jnp.float32)
        t2 = jax.lax.dot_general(x, muiv_ref[:], _DN_T,
                                 preferred_element_type=jnp.float32)
        d2 = jnp.maximum(t1 - 2.0 * t2 + t3_ref[:], 0.0)
        dmin = _min_lanes(d2)
        g = jnp.exp(-0.5 * (d2 - dmin))

        iota = jax.lax.broadcasted_iota(jnp.int32, (BB, CP), 1)
        cc = _first_index_where(g == 1.0, iota)
        gm = jnp.where(labels_ref[:] == assign_ref[:], g, 0.0)
        mg = _max_lanes(gm)
        jc = _first_index_where(gm == mg, iota)

        s = jnp.sum(_fold_lanes(g, jnp.add), axis=1, keepdims=True)
        gn = g / (s + 1e-12)
        scores = jax.lax.dot_general(gn, claug_ref[:], _DN_ROW,
                                     preferred_element_type=jnp.float32)
        m = jnp.max(scores, axis=1, keepdims=True)
        kidx = jax.lax.broadcasted_iota(jnp.int32, scores.shape, 1)
        pc = jnp.min(jnp.where(scores == m, kidx, _BIG), axis=1, keepdims=True)

        onehot = (jc == iota).astype(jnp.float32)
        count_ref[:] += jax.lax.dot_general(
            jnp.ones((1, BB), jnp.float32), onehot, _DN_ROW,
            preferred_element_type=jnp.float32)
        j_ref[pl.ds(b * BB, BB), :] = jc
        scores_ref[pl.ds(b * BB, BB), :] = scores
        pred_ref[pl.ds(b * BB, BB)] = pc[:, 0]
        clusters_ref[pl.ds(b * BB, BB)] = cc[:, 0]

    @pl.when(i == NBLK)
    def _init2():
        nn = n_ref[:] + count_ref[0, :]
        nnew_ref[:] = nn
        muaug_ref[:, :D] = mu_ref[:]
        muaug_ref[:, D:] = jnp.broadcast_to(nn[:, None], (CP, D))
        munew_ref[:] = mu_ref[:]
        snew_ref[:] = s_ref[:]

    @pl.when(i >= NBLK)
    def _update():
        b = i - NBLK
        x = data_ref[:]
        jc = j_ref[pl.ds(b * BB, BB), :]
        iota = jax.lax.broadcasted_iota(jnp.int32, (BB, CP), 1)
        p = (jc == iota).astype(jnp.float32)
        gath = jax.lax.dot_general(p, muaug_ref[:], _DN_ROW,
                                   preferred_element_type=jnp.float32)
        e = x - gath[:, :D]
        w = 1.0 / gath[:, D:D + 1]
        upd = jnp.concatenate([e * w, e * e], axis=1)
        delta = jax.lax.dot_general(p, upd, _DN_COL,
                                    preferred_element_type=jnp.float32)
        munew_ref[:] += delta[:, :D]
        snew_ref[:] += delta[:, D:]


def kernel(data, labels, n, mu, S_diag, cluster_labels):
    pad = CP - C
    mu_p = jnp.pad(mu, ((0, pad), (0, 0)))
    s_p = jnp.pad(S_diag, ((0, pad), (0, 0)))
    n_p = jnp.pad(n, (0, pad), constant_values=1.0)
    cl_p = jnp.pad(cluster_labels, ((0, pad), (0, 0)))
    labels_col = labels[:, None]

    out_shapes = (
        jax.ShapeDtypeStruct((B, K), jnp.float32),
        jax.ShapeDtypeStruct((B,), jnp.int32),
        jax.ShapeDtypeStruct((B,), jnp.int32),
        jax.ShapeDtypeStruct((CP,), jnp.float32),
        jax.ShapeDtypeStruct((CP, D), jnp.float32),
        jax.ShapeDtypeStruct((CP, D), jnp.float32),
    )
    blk = lambda i: (jnp.where(i < NBLK, i, i - NBLK), 0)
    in_specs = [
        pl.BlockSpec((BB, D), blk),
        pl.BlockSpec((BB, 1), blk),
        pl.BlockSpec((CP,), lambda i: (0,)),
        pl.BlockSpec((CP, D), lambda i: (0, 0)),
        pl.BlockSpec((CP, D), lambda i: (0, 0)),
        pl.BlockSpec((CP, K), lambda i: (0, 0)),
    ]
    out_specs = (
        pl.BlockSpec((B, K), lambda i: (0, 0)),
        pl.BlockSpec((B,), lambda i: (0,)),
        pl.BlockSpec((B,), lambda i: (0,)),
        pl.BlockSpec((CP,), lambda i: (0,)),
        pl.BlockSpec((CP, D), lambda i: (0, 0)),
        pl.BlockSpec((CP, D), lambda i: (0, 0)),
    )
    scratch = [
        pltpu.VMEM((CP, D), jnp.float32),
        pltpu.VMEM((CP, D), jnp.float32),
        pltpu.VMEM((1, CP), jnp.float32),
        pltpu.VMEM((1, CP), jnp.int32),
        pltpu.VMEM((CP, K), jnp.float32),
        pltpu.VMEM((1, CP), jnp.float32),
        pltpu.VMEM((B, 1), jnp.int32),
        pltpu.VMEM((CP, 2 * D), jnp.float32),
    ]
    scores, pred, clusters, n_new, mu_new, S_new = pl.pallas_call(
        _body, grid=(2 * NBLK,), in_specs=in_specs, out_specs=out_specs,
        out_shape=out_shapes, scratch_shapes=scratch,
    )(data, labels_col, n_p, mu_p, s_p, cl_p)
    return (scores, pred, clusters, n_new[:C], mu_new[:C], S_new[:C])


# speedup vs baseline: 1.0121x; 1.0121x over previous
import jax
import jax.numpy as jnp
from jax.experimental import pallas as pl
from jax.experimental.pallas import tpu as pltpu

B = 1024
D = 128
C = 2000
CP = 2048
K = 10
BB = 256
NBLK = B // BB
NCH = CP // 128

_DN_T = (((1,), (1,)), ((), ()))
_DN_ROW = (((1,), (0,)), ((), ()))
_DN_COL = (((0,), (0,)), ((), ()))
_BIG = jnp.iinfo(jnp.int32).max


def _fold_lanes(x, op):
    m = x[:, 0:128]
    for k in range(1, NCH):
        m = op(m, x[:, k * 128:(k + 1) * 128])
    return m


def _min_lanes(x):
    return jnp.min(_fold_lanes(x, jnp.minimum), axis=1, keepdims=True)


def _max_lanes(x):
    return jnp.max(_fold_lanes(x, jnp.maximum), axis=1, keepdims=True)


def _first_index_where(cond, iota):
    cand = None
    for k in range(NCH):
        sl = slice(k * 128, (k + 1) * 128)
        c = jnp.where(cond[:, sl], iota[:, sl], _BIG)
        cand = c if cand is None else jnp.minimum(cand, c)
    return jnp.min(cand, axis=1, keepdims=True)


def _body(data_ref, labels_ref, n_ref, mu_ref, s_ref, cl_ref,
          scores_ref, pred_ref, clusters_ref, nnew_ref, munew_ref, snew_ref,
          iv_ref, muiv_ref, t3_ref, assign_ref, claug_ref, count_ref,
          j_ref, muaug_ref):
    i = pl.program_id(0)

    @pl.when(i == 0)
    def _init():
        var = s_ref[:] / jnp.maximum(n_ref[:], 1.0)[:, None] + 1e-6
        iv = 1.0 / var
        iv_ref[:] = iv
        muiv_ref[:] = (2.0 * mu_ref[:]) * iv
        t3_ref[:] = jnp.sum(mu_ref[:] * mu_ref[:] * iv, axis=1)[None, :]
        cl = cl_ref[:]
        cidx = jax.lax.broadcasted_iota(jnp.int32, cl.shape, 1)
        assign_ref[:] = jnp.sum(cl * cidx, axis=1)[None, :]
        claug_ref[:] = cl.astype(jnp.float32)
        count_ref[:] = jnp.zeros_like(count_ref)

    @pl.when(i < NBLK)
    def _activation():
        b = i
        x = data_ref[:]
        t1 = jax.lax.dot_general(x * x, iv_ref[:], _DN_T,
                                 preferred_element_type=jnp.float32)
        t2 = jax.lax.dot_general(x, muiv_ref[:], _DN_T,
                                 preferred_element_type=jnp.float32)
        d2 = jnp.maximum(t1 - t2 + t3_ref[:], 0.0)
        dmin = _min_lanes(d2)
        g = jnp.exp(-0.5 * (d2 - dmin))

        iota = jax.lax.broadcasted_iota(jnp.int32, (BB, CP), 1)
        cc = _first_index_where(g == 1.0, iota)
        gm = jnp.where(labels_ref[:] == assign_ref[:], g, 0.0)
        mg = _max_lanes(gm)
        jc = _first_index_where(gm == mg, iota)

        s = jnp.sum(_fold_lanes(g, jnp.add), axis=1, keepdims=True)
        gn = g / (s + 1e-12)
        scores = jax.lax.dot_general(gn, claug_ref[:], _DN_ROW,
                                     preferred_element_type=jnp.float32)
        m = jnp.max(scores, axis=1, keepdims=True)
        kidx = jax.lax.broadcasted_iota(jnp.int32, scores.shape, 1)
        pc = jnp.min(jnp.where(scores == m, kidx, _BIG), axis=1, keepdims=True)

        onehot = (jc == iota).astype(jnp.bfloat16)
        count_ref[:] += jax.lax.dot_general(
            jnp.ones((1, BB), jnp.bfloat16), onehot, _DN_ROW,
            preferred_element_type=jnp.float32)
        j_ref[pl.ds(b * BB, BB), :] = jc
        scores_ref[pl.ds(b * BB, BB), :] = scores
        pred_ref[pl.ds(b * BB, BB)] = pc[:, 0]
        clusters_ref[pl.ds(b * BB, BB)] = cc[:, 0]

    @pl.when(i == NBLK)
    def _init2():
        nn = n_ref[:] + count_ref[0, :]
        nnew_ref[:] = nn
        muaug_ref[:, :D] = mu_ref[:].astype(jnp.bfloat16)
        muaug_ref[:, D:] = jnp.broadcast_to(
            nn[:, None].astype(jnp.bfloat16), (CP, D))
        munew_ref[:] = mu_ref[:]
        snew_ref[:] = s_ref[:]

    @pl.when(i >= NBLK)
    def _update():
        b = i - NBLK
        x = data_ref[:]
        jc = j_ref[pl.ds(b * BB, BB), :]
        iota = jax.lax.broadcasted_iota(jnp.int32, (BB, CP), 1)
        p = (jc == iota).astype(jnp.bfloat16)
        gath = jax.lax.dot_general(p, muaug_ref[:], _DN_ROW,
                                   preferred_element_type=jnp.float32)
        e = x - gath[:, :D]
        w = 1.0 / gath[:, D:D + 1]
        upd = jnp.concatenate([e * w, e * e], axis=1).astype(jnp.bfloat16)
        delta = jax.lax.dot_general(p, upd, _DN_COL,
                                    preferred_element_type=jnp.float32)
        munew_ref[:] += delta[:, :D]
        snew_ref[:] += delta[:, D:]


def kernel(data, labels, n, mu, S_diag, cluster_labels):
    pad = CP - C
    mu_p = jnp.pad(mu, ((0, pad), (0, 0)))
    s_p = jnp.pad(S_diag, ((0, pad), (0, 0)))
    n_p = jnp.pad(n, (0, pad), constant_values=1.0)
    cl_p = jnp.pad(cluster_labels, ((0, pad), (0, 0)))
    labels_col = labels[:, None]

    out_shapes = (
        jax.ShapeDtypeStruct((B, K), jnp.float32),
        jax.ShapeDtypeStruct((B,), jnp.int32),
        jax.ShapeDtypeStruct((B,), jnp.int32),
        jax.ShapeDtypeStruct((CP,), jnp.float32),
        jax.ShapeDtypeStruct((CP, D), jnp.float32),
        jax.ShapeDtypeStruct((CP, D), jnp.float32),
    )
    blk = lambda i: (jnp.where(i < NBLK, i, i - NBLK), 0)
    in_specs = [
        pl.BlockSpec((BB, D), blk),
        pl.BlockSpec((BB, 1), blk),
        pl.BlockSpec((CP,), lambda i: (0,)),
        pl.BlockSpec((CP, D), lambda i: (0, 0)),
        pl.BlockSpec((CP, D), lambda i: (0, 0)),
        pl.BlockSpec((CP, K), lambda i: (0, 0)),
    ]
    out_specs = (
        pl.BlockSpec((B, K), lambda i: (0, 0)),
        pl.BlockSpec((B,), lambda i: (0,)),
        pl.BlockSpec((B,), lambda i: (0,)),
        pl.BlockSpec((CP,), lambda i: (0,)),
        pl.BlockSpec((CP, D), lambda i: (0, 0)),
        pl.BlockSpec((CP, D), lambda i: (0, 0)),
    )
    scratch = [
        pltpu.VMEM((CP, D), jnp.float32),
        pltpu.VMEM((CP, D), jnp.float32),
        pltpu.VMEM((1, CP), jnp.float32),
        pltpu.VMEM((1, CP), jnp.int32),
        pltpu.VMEM((CP, K), jnp.float32),
        pltpu.VMEM((1, CP), jnp.float32),
        pltpu.VMEM((B, 1), jnp.int32),
        pltpu.VMEM((CP, 2 * D), jnp.bfloat16),
    ]
    scores, pred, clusters, n_new, mu_new, S_new = pl.pallas_call(
        _body, grid=(2 * NBLK,), in_specs=in_specs, out_specs=out_specs,
        out_shape=out_shapes, scratch_shapes=scratch,
    )(data, labels_col, n_p, mu_p, s_p, cl_p)
    return (scores, pred, clusters, n_new[:C], mu_new[:C], S_new[:C])
